# baseline (speedup 1.0000x reference)
import functools

import jax
import jax.numpy as jnp
from jax import lax
from jax.experimental import pallas as pl
from jax.experimental.pallas import tpu as pltpu
from jax.experimental.pallas import tpu_sc as plsc

N_NODES = 10000
LANES = 16
NUM_CORES = 2
NUM_SUBCORES = 16
NUM_WORKERS = NUM_CORES * NUM_SUBCORES
NPW = 320


@functools.partial(jax.jit, static_argnums=(2, 3))
def _mean_agg(neighs_t, table, k_nb, d_feat):
    inv_k = jnp.float32(1.0 / k_nb)

    mesh = plsc.VectorSubcoreMesh(
        core_axis_name="c", subcore_axis_name="s", num_cores=NUM_CORES,
        num_subcores=NUM_SUBCORES)

    @functools.partial(
        pl.kernel,
        out_type=jax.ShapeDtypeStruct((N_NODES, d_feat), jnp.float32),
        mesh=mesh,
        scratch_types=[
            pltpu.VMEM((k_nb * NPW,), jnp.int32),
            pltpu.VMEM((NPW, d_feat), jnp.float32),
            pltpu.VMEM((NPW, d_feat), jnp.float32),
            pltpu.SemaphoreType.DMA,
        ],
    )
    def k(neighs_hbm, table_hbm, out_hbm, idxt_v, acc_a, acc_b, sem):
        wid = lax.axis_index("s") * NUM_CORES + lax.axis_index("c")
        start = jnp.minimum(wid * NPW, N_NODES - NPW)

        def stage_body(j, _):
            pltpu.async_copy(neighs_hbm.at[pl.ds(j * N_NODES + start, NPW)],
                             idxt_v.at[pl.ds(j * NPW, NPW)], sem)
            return 0
        lax.fori_loop(0, k_nb, stage_body, 0)

        zeros = jnp.zeros((LANES,), jnp.float32)

        def zero_body(c, _):
            for d in range(d_feat // LANES):
                acc_a[c, pl.ds(d * LANES, LANES)] = zeros
                acc_b[c, pl.ds(d * LANES, LANES)] = zeros
            return 0
        lax.fori_loop(0, NPW, zero_body, 0)

        def stage_drain(j, _):
            pltpu.make_async_copy(
                neighs_hbm.at[pl.ds(0, NPW)],
                idxt_v.at[pl.ds(0, NPW)], sem).wait()
            return 0
        lax.fori_loop(0, k_nb, stage_drain, 0)

        def fire_body(j, _):
            pltpu.async_copy(
                table_hbm.at[idxt_v.at[pl.ds((2 * j) * NPW, NPW)]],
                acc_a, sem, add=True)
            pltpu.async_copy(
                table_hbm.at[idxt_v.at[pl.ds((2 * j + 1) * NPW, NPW)]],
                acc_b, sem, add=True)
            return 0
        lax.fori_loop(0, k_nb // 2, fire_body, 0)

        def drain_body(j, _):
            pltpu.make_async_copy(table_hbm.at[idxt_v.at[pl.ds(0, NPW)]],
                                  acc_a, sem).wait()
            return 0
        lax.fori_loop(0, k_nb, drain_body, 0)

        for d in range(d_feat // LANES):
            sl = pl.ds(d * LANES, LANES)

            def scale_body(c, _):
                acc_a[c, sl] = (acc_a[c, sl] + acc_b[c, sl]) * inv_k
                return 0
            lax.fori_loop(0, NPW, scale_body, 0)
        pltpu.sync_copy(acc_a, out_hbm.at[pl.ds(start, NPW)])

    return k(neighs_t, table)


def kernel(neighs, node_count, table):
    del node_count
    k_nb = neighs.shape[0] // N_NODES
    neighs_t = neighs.astype(jnp.int32).reshape(N_NODES, k_nb).T.reshape(-1)
    return _mean_agg(neighs_t, table, k_nb, table.shape[1])


# speedup vs baseline: 1.0327x; 1.0327x over previous
import functools

import jax
import jax.numpy as jnp
from jax import lax
from jax.experimental import pallas as pl
from jax.experimental.pallas import tpu as pltpu
from jax.experimental.pallas import tpu_sc as plsc

N_NODES = 10000
LANES = 16
NUM_CORES = 2
NUM_SUBCORES = 16
NUM_WORKERS = NUM_CORES * NUM_SUBCORES
NPW = 320


@functools.partial(jax.jit, static_argnums=(2, 3))
def _mean_agg(neighs_t, table, k_nb, d_feat):
    inv_k = jnp.float32(1.0 / k_nb)

    mesh = plsc.VectorSubcoreMesh(
        core_axis_name="c", subcore_axis_name="s", num_cores=NUM_CORES,
        num_subcores=NUM_SUBCORES)

    @functools.partial(
        pl.kernel,
        out_type=jax.ShapeDtypeStruct((N_NODES, d_feat), jnp.float32),
        mesh=mesh,
        compiler_params=pltpu.CompilerParams(use_tc_tiling_on_sc=False),
        scratch_types=[
            pltpu.VMEM((k_nb * NPW,), jnp.int32),
            pltpu.VMEM((NPW, d_feat), jnp.float32),
            pltpu.SemaphoreType.DMA,
        ],
    )
    def k(neighs_hbm, table_hbm, out_hbm, idxt_v, acc_v, sem):
        wid = lax.axis_index("s") * NUM_CORES + lax.axis_index("c")
        start = jnp.minimum(wid * NPW, N_NODES - NPW)

        def stage_body(j, _):
            pltpu.async_copy(neighs_hbm.at[pl.ds(j * N_NODES + start, NPW)],
                             idxt_v.at[pl.ds(j * NPW, NPW)], sem)
            return 0
        lax.fori_loop(0, k_nb, stage_body, 0)

        zeros = jnp.zeros((LANES,), jnp.float32)

        def zero_body(c, _):
            for d in range(d_feat // LANES):
                acc_v[c, pl.ds(d * LANES, LANES)] = zeros
            return 0
        lax.fori_loop(0, NPW, zero_body, 0)

        def stage_drain(j, _):
            pltpu.make_async_copy(
                neighs_hbm.at[pl.ds(0, NPW)],
                idxt_v.at[pl.ds(0, NPW)], sem).wait()
            return 0
        lax.fori_loop(0, k_nb, stage_drain, 0)

        def fire_body(j, _):
            pltpu.async_copy(table_hbm.at[idxt_v.at[pl.ds(j * NPW, NPW)]],
                             acc_v, sem, add=True)
            return 0
        lax.fori_loop(0, k_nb, fire_body, 0)

        def drain_body(j, _):
            pltpu.make_async_copy(table_hbm.at[idxt_v.at[pl.ds(0, NPW)]],
                                  acc_v, sem).wait()
            return 0
        lax.fori_loop(0, k_nb, drain_body, 0)

        for d in range(d_feat // LANES):
            sl = pl.ds(d * LANES, LANES)

            def scale_body(c, _):
                acc_v[c, sl] = acc_v[c, sl] * inv_k
                return 0
            lax.fori_loop(0, NPW, scale_body, 0)
        pltpu.sync_copy(acc_v, out_hbm.at[pl.ds(start, NPW)])

    return k(neighs_t, table)


def kernel(neighs, node_count, table):
    del node_count
    k_nb = neighs.shape[0] // N_NODES
    neighs_t = neighs.astype(jnp.int32).reshape(N_NODES, k_nb).T.reshape(-1)
    return _mean_agg(neighs_t, table, k_nb, table.shape[1])


# speedup vs baseline: 1.0347x; 1.0019x over previous
import functools

import jax
import jax.numpy as jnp
from jax import lax
from jax.experimental import pallas as pl
from jax.experimental.pallas import tpu as pltpu
from jax.experimental.pallas import tpu_sc as plsc

N_NODES = 10000
LANES = 16
NUM_CORES = 2
NUM_SUBCORES = 16
NUM_WORKERS = NUM_CORES * NUM_SUBCORES
NPW = 320


@functools.partial(jax.jit, static_argnums=(2, 3))
def _mean_agg(neighs_t, table, k_nb, d_feat):
    inv_k = jnp.float32(1.0 / k_nb)

    mesh = plsc.VectorSubcoreMesh(
        core_axis_name="c", subcore_axis_name="s", num_cores=NUM_CORES,
        num_subcores=NUM_SUBCORES)

    @functools.partial(
        pl.kernel,
        out_type=jax.ShapeDtypeStruct((N_NODES, d_feat), jnp.float32),
        mesh=mesh,
        scratch_types=[
            pltpu.VMEM((k_nb * NPW,), jnp.int32),
            pltpu.VMEM((NPW, d_feat), jnp.float32),
            pltpu.SemaphoreType.DMA,
        ],
    )
    def k(neighs_hbm, table_hbm, out_hbm, idxt_v, acc_v, sem):
        wid = lax.axis_index("s") * NUM_CORES + lax.axis_index("c")
        start = jnp.minimum(wid * NPW, N_NODES - NPW)

        def stage_body(j, _):
            pltpu.async_copy(neighs_hbm.at[pl.ds(j * N_NODES + start, NPW)],
                             idxt_v.at[pl.ds(j * NPW, NPW)], sem)
            return 0
        lax.fori_loop(0, k_nb, stage_body, 0)

        zeros = jnp.zeros((LANES,), jnp.float32)

        def zero_body(c, _):
            for d in range(d_feat // LANES):
                acc_v[c, pl.ds(d * LANES, LANES)] = zeros
            return 0
        lax.fori_loop(0, NPW, zero_body, 0)

        def stage_drain(j, _):
            pltpu.make_async_copy(
                neighs_hbm.at[pl.ds(0, NPW)],
                idxt_v.at[pl.ds(0, NPW)], sem).wait()
            return 0
        lax.fori_loop(0, k_nb, stage_drain, 0)

        def fire_body(j, _):
            pltpu.async_copy(table_hbm.at[idxt_v.at[pl.ds(j * NPW, NPW)]],
                             acc_v, sem, add=True)
            return 0
        lax.fori_loop(0, k_nb, fire_body, 0)

        def drain_body(j, _):
            pltpu.make_async_copy(table_hbm.at[idxt_v.at[pl.ds(0, NPW)]],
                                  acc_v, sem).wait()
            return 0
        lax.fori_loop(0, k_nb, drain_body, 0)

        for d in range(d_feat // LANES):
            sl = pl.ds(d * LANES, LANES)

            def scale_body(c, _):
                acc_v[c, sl] = acc_v[c, sl] * inv_k
                return 0
            lax.fori_loop(0, NPW, scale_body, 0)
        pltpu.sync_copy(acc_v, out_hbm.at[pl.ds(start, NPW)])

    return k(neighs_t, table)


def kernel(neighs, node_count, table):
    del node_count
    k_nb = neighs.shape[0] // N_NODES
    neighs_t = neighs.astype(jnp.int32).reshape(N_NODES, k_nb).T.reshape(-1)
    return _mean_agg(neighs_t, table, k_nb, table.shape[1])


# speedup vs baseline: 1.0448x; 1.0098x over previous
import functools

import jax
import jax.numpy as jnp
from jax import lax
from jax.experimental import pallas as pl
from jax.experimental.pallas import tpu as pltpu
from jax.experimental.pallas import tpu_sc as plsc

N_NODES = 10000
LANES = 16
NUM_CORES = 2
NUM_SUBCORES = 16
NUM_WORKERS = NUM_CORES * NUM_SUBCORES
NPW = 320


@functools.partial(jax.jit, static_argnums=(2, 3))
def _mean_agg(neighs, table, k_nb, d_feat):
    inv_k = jnp.float32(1.0 / k_nb)
    epw = k_nb * NPW

    mesh = plsc.VectorSubcoreMesh(
        core_axis_name="c", subcore_axis_name="s", num_cores=NUM_CORES,
        num_subcores=NUM_SUBCORES)

    @functools.partial(
        pl.kernel,
        out_type=jax.ShapeDtypeStruct((N_NODES, d_feat), jnp.float32),
        mesh=mesh,
        compiler_params=pltpu.CompilerParams(needs_layout_passes=False),
        scratch_types=[
            pltpu.VMEM((epw,), jnp.int32),
            pltpu.VMEM((epw,), jnp.int32),
            pltpu.VMEM((NPW, d_feat), jnp.float32),
            pltpu.SemaphoreType.DMA,
        ],
    )
    def k(neighs_hbm, table_hbm, out_hbm, idx_nm, idxt_v, acc_v, sem):
        wid = lax.axis_index("s") * NUM_CORES + lax.axis_index("c")
        start = jnp.minimum(wid * NPW, N_NODES - NPW)

        pltpu.async_copy(neighs_hbm.at[pl.ds(start * k_nb, epw)], idx_nm, sem)

        zeros = jnp.zeros((LANES,), jnp.float32)

        def zero_body(c, _):
            for d in range(d_feat // LANES):
                acc_v[c, pl.ds(d * LANES, LANES)] = zeros
            return 0
        lax.fori_loop(0, NPW, zero_body, 0)

        pltpu.make_async_copy(neighs_hbm.at[pl.ds(0, epw)], idx_nm, sem).wait()

        lanes_npw = lax.broadcasted_iota(jnp.int32, (LANES,), 0) * NPW

        def t_body(e0, _):
            vec = idx_nm[pl.ds(e0 * LANES, LANES)]
            base = (e0 % 2) * (LANES * NPW) + e0 // 2
            plsc.store_scatter(idxt_v, [lanes_npw + base], vec)
            return 0
        lax.fori_loop(0, epw // LANES, t_body, 0)

        def fire_body(j, _):
            pltpu.async_copy(table_hbm.at[idxt_v.at[pl.ds(j * NPW, NPW)]],
                             acc_v, sem, add=True)
            return 0
        lax.fori_loop(0, k_nb, fire_body, 0)

        def drain_body(j, _):
            pltpu.make_async_copy(table_hbm.at[idxt_v.at[pl.ds(0, NPW)]],
                                  acc_v, sem).wait()
            return 0
        lax.fori_loop(0, k_nb, drain_body, 0)

        for d in range(d_feat // LANES):
            sl = pl.ds(d * LANES, LANES)

            def scale_body(c, _):
                acc_v[c, sl] = acc_v[c, sl] * inv_k
                return 0
            lax.fori_loop(0, NPW, scale_body, 0)
        pltpu.sync_copy(acc_v, out_hbm.at[pl.ds(start, NPW)])

    return k(neighs, table)


def kernel(neighs, node_count, table):
    del node_count
    k_nb = neighs.shape[0] // N_NODES
    return _mean_agg(neighs.astype(jnp.int32), table, k_nb, table.shape[1])


# speedup vs baseline: 1.1722x; 1.1220x over previous
import functools

import jax
import jax.numpy as jnp
from jax import lax
from jax.experimental import pallas as pl
from jax.experimental.pallas import tpu as pltpu
from jax.experimental.pallas import tpu_sc as plsc

N_NODES = 10000
LANES = 16
NUM_CORES = 2
NUM_SUBCORES = 16
NUM_WORKERS = NUM_CORES * NUM_SUBCORES
NPW = 320


@functools.partial(jax.jit, static_argnums=(2, 3))
def _mean_agg(neighs, table, k_nb, d_feat):
    inv_k = jnp.float32(1.0 / k_nb)
    epw = k_nb * NPW

    mesh = plsc.VectorSubcoreMesh(
        core_axis_name="c", subcore_axis_name="s", num_cores=NUM_CORES,
        num_subcores=NUM_SUBCORES)

    @functools.partial(
        pl.kernel,
        out_type=jax.ShapeDtypeStruct((N_NODES, d_feat), jnp.float32),
        mesh=mesh,
        compiler_params=pltpu.CompilerParams(needs_layout_passes=False),
        scratch_types=[
            pltpu.VMEM((epw,), jnp.int32),
            pltpu.VMEM((epw,), jnp.int32),
            pltpu.VMEM((NPW, d_feat), jnp.float32),
            pltpu.SemaphoreType.DMA,
        ],
    )
    def k(neighs_hbm, table_hbm, out_hbm, idx_nm, idxt_v, acc_v, sem):
        wid = lax.axis_index("s") * NUM_CORES + lax.axis_index("c")
        start = jnp.minimum(wid * NPW, N_NODES - NPW)

        pltpu.async_copy(neighs_hbm.at[pl.ds(start * k_nb, epw)], idx_nm, sem)

        zeros = jnp.zeros((LANES,), jnp.float32)

        def zero_body(c, _):
            for d in range(d_feat // LANES):
                acc_v[c, pl.ds(d * LANES, LANES)] = zeros
            return 0
        lax.fori_loop(0, NPW, zero_body, 0, unroll=8)

        pltpu.make_async_copy(neighs_hbm.at[pl.ds(0, epw)], idx_nm, sem).wait()

        lanes_npw = lax.broadcasted_iota(jnp.int32, (LANES,), 0) * NPW

        def t_body(e0, _):
            vec = idx_nm[pl.ds(e0 * LANES, LANES)]
            base = (e0 % 2) * (LANES * NPW) + e0 // 2
            plsc.store_scatter(idxt_v, [lanes_npw + base], vec)
            return 0
        lax.fori_loop(0, epw // LANES, t_body, 0, unroll=8)

        def fire_body(j, _):
            pltpu.async_copy(table_hbm.at[idxt_v.at[pl.ds(j * NPW, NPW)]],
                             acc_v, sem, add=True)
            return 0
        lax.fori_loop(0, k_nb, fire_body, 0)

        def drain_body(j, _):
            pltpu.make_async_copy(table_hbm.at[idxt_v.at[pl.ds(0, NPW)]],
                                  acc_v, sem).wait()
            return 0
        lax.fori_loop(0, k_nb, drain_body, 0)

        for d in range(d_feat // LANES):
            sl = pl.ds(d * LANES, LANES)

            def scale_body(c, _):
                acc_v[c, sl] = acc_v[c, sl] * inv_k
                return 0
            lax.fori_loop(0, NPW, scale_body, 0, unroll=8)
        pltpu.sync_copy(acc_v, out_hbm.at[pl.ds(start, NPW)])

    return k(neighs, table)


def kernel(neighs, node_count, table):
    del node_count
    k_nb = neighs.shape[0] // N_NODES
    return _mean_agg(neighs.astype(jnp.int32), table, k_nb, table.shape[1])
